# baseline (speedup 1.0000x reference)
import functools

import jax
import jax.numpy as jnp
from jax import lax
from jax.experimental import pallas as pl
from jax.experimental.pallas import tpu as pltpu
from jax.experimental.pallas import tpu_sc as plsc

NC = 2
NS = 16
L = 16
C = 80
CORE0_FRAC = 0.77
NIDX = 4
NBUF = 2


def _matmul_body(x_ref, w_ref, o_ref):
    o_ref[...] = jnp.dot(x_ref[...], w_ref[...], preferred_element_type=jnp.float32)


def _combine_body(p_ref, b_ref, o_ref):
    n = o_ref.shape[0]
    o_ref[...] = p_ref[0, :n] + p_ref[1, :n] + b_ref[...]


@functools.lru_cache(maxsize=None)
def _make_sc_scatter(n, d, ka, kb):
    assert ka % NIDX == 0 and kb % NIDX == 0 and min(ka, kb) >= NIDX
    rpt = n // NS
    nfull, rem = divmod(rpt, C)
    mesh = plsc.VectorSubcoreMesh(core_axis_name="c", subcore_axis_name="s")

    @functools.partial(
        pl.kernel,
        out_type=jax.ShapeDtypeStruct((NC, n, d), jnp.float32),
        mesh=mesh,
        scratch_types=[
            pltpu.VMEM((NIDX, 2, C), jnp.int32),
            pltpu.VMEM((NIDX, C), jnp.float32),
            pltpu.VMEM((NBUF, C, d), jnp.float32),
            pltpu.VMEM((NBUF, C, d), jnp.float32),
            pltpu.VMEM((NBUF, C), jnp.int32),
            pltpu.VMEM_SHARED((n, d), jnp.float32),
            [pltpu.SemaphoreType.DMA] * NIDX,
            [pltpu.SemaphoreType.DMA] * NIDX,
            [pltpu.SemaphoreType.DMA] * NBUF,
            [pltpu.SemaphoreType.DMA] * NBUF,
        ],
    )
    def sc_kernel(
        support, pk, wk, out, idx_v, w_v, gbuf, sbuf, dst_v, acc, sem_i, sem_w,
        sem_g, sem_s
    ):
        cid = lax.axis_index("c")
        sid = lax.axis_index("s")
        kc = jnp.where(cid == 0, ka, kb)
        row0 = jnp.where(cid == 0, sid * ka, NS * ka + sid * kb)

        def idx_start(b, j):
            pltpu.async_copy(pk.at[row0 + j], idx_v.at[b], sem_i[b])
            pltpu.async_copy(wk.at[row0 + j], w_v.at[b], sem_w[b])

        def idx_wait(b):
            pltpu.make_async_copy(pk.at[row0], idx_v.at[b], sem_i[b]).wait()
            pltpu.make_async_copy(wk.at[row0], w_v.at[b], sem_w[b]).wait()

        def gather_start(gs, b):
            pltpu.async_copy(support.at[idx_v.at[b, 0]], gbuf.at[gs], sem_g[gs])

        def gather_wait(gs):
            pltpu.make_async_copy(
                support.at[idx_v.at[0, 0]], gbuf.at[gs], sem_g[gs]
            ).wait()

        def scatter_start(gs):
            pltpu.async_copy(
                sbuf.at[gs], acc.at[dst_v.at[gs]], sem_s[gs], add=True
            )

        def scatter_wait(gs):
            pltpu.make_async_copy(
                sbuf.at[gs], acc.at[dst_v.at[gs]], sem_s[gs]
            ).wait()

        def scale(b, gs):
            @pl.loop(0, C // L)
            def _(blk):
                sl16 = pl.ds(blk * L, L)
                wv16 = w_v[b, sl16]
                dst_v[gs, sl16] = idx_v[b, 1, sl16]
                for r in range(L):
                    wvec = jnp.full((L,), wv16[r])
                    row = blk * L + r
                    for jj in range(d // L):
                        sl = pl.ds(jj * L, L)
                        sbuf[gs, row, sl] = gbuf[gs, row, sl] * wvec

        for b in range(NIDX):
            idx_start(b, b)

        zero = jnp.zeros((L,), jnp.float32)

        @pl.loop(0, C)
        def _(i):
            for j in range(d // L):
                gbuf[0, i, pl.ds(j * L, L)] = zero

        base = sid * rpt
        for t in range(nfull):
            pltpu.sync_copy(gbuf.at[0], acc.at[pl.ds(base + t * C, C)])
        if rem:
            pltpu.sync_copy(
                gbuf.at[0, pl.ds(0, rem)], acc.at[pl.ds(base + nfull * C, rem)]
            )
        plsc.subcore_barrier()

        for b in range(NBUF):
            idx_wait(b)
            gather_start(b, b)

        @pl.loop(0, kc, step=NIDX)
        def _(j0):
            for b in range(NIDX):
                j = j0 + b
                gs = b % NBUF
                gather_wait(gs)
                if b < NBUF:
                    @pl.when(j >= NBUF)
                    def _():
                        scatter_wait(gs)
                else:
                    scatter_wait(gs)
                scale(b, gs)
                scatter_start(gs)
                @pl.when(j + NBUF < kc)
                def _():
                    idx_wait((b + NBUF) % NIDX)
                    gather_start(gs, (b + NBUF) % NIDX)
                @pl.when(j + NIDX < kc)
                def _():
                    idx_start(b, j + NIDX)

        for gs in range(NBUF):
            scatter_wait(gs)
        plsc.subcore_barrier()

        for t in range(nfull):
            pltpu.sync_copy(
                acc.at[pl.ds(base + t * C, C)], out.at[cid, pl.ds(base + t * C, C)]
            )
        if rem:
            pltpu.sync_copy(
                acc.at[pl.ds(base + nfull * C, rem)],
                out.at[cid, pl.ds(base + nfull * C, rem)],
            )

    return sc_kernel


def kernel(input_feature, edge_index, edge_weight, weight, bias):
    n, _ = input_feature.shape
    d_out = weight.shape[1]
    e = edge_weight.shape[0]

    support = pl.pallas_call(
        _matmul_body,
        out_shape=jax.ShapeDtypeStruct((n, d_out), jnp.float32),
    )(input_feature, weight)

    nw = NC * NS
    k = -(-e // (nw * C))
    k = -(-k // NIDX) * NIDX
    ka = int(round(2 * k * CORE0_FRAC / NIDX)) * NIDX
    ka = min(max(ka, NIDX), 2 * k - NIDX)
    kb = 2 * k - ka
    epad = nw * k * C
    dst = edge_index[0].astype(jnp.int32)
    src = edge_index[1].astype(jnp.int32)
    ew = edge_weight.astype(jnp.float32)
    if epad != e:
        pad = epad - e
        dst = jnp.concatenate([dst, jnp.zeros((pad,), jnp.int32)])
        src = jnp.concatenate([src, jnp.zeros((pad,), jnp.int32)])
        ew = jnp.concatenate([ew, jnp.zeros((pad,), jnp.float32)])
    pk = jnp.concatenate(
        [src.reshape(nw * k, 1, C), dst.reshape(nw * k, 1, C)], axis=1
    )
    wk = ew.reshape(nw * k, C)

    rpt = -(-n // NS)
    rpt = (rpt + 7) // 8 * 8
    n_pad = rpt * NS

    partial = _make_sc_scatter(n_pad, d_out, ka, kb)(support, pk, wk)

    out = pl.pallas_call(
        _combine_body,
        out_shape=jax.ShapeDtypeStruct((n, d_out), jnp.float32),
    )(partial, bias.reshape(1, d_out))
    return out


# speedup vs baseline: 1.0092x; 1.0092x over previous
import functools

import jax
import jax.numpy as jnp
from jax import lax
from jax.experimental import pallas as pl
from jax.experimental.pallas import tpu as pltpu
from jax.experimental.pallas import tpu_sc as plsc

NC = 2
NS = 16
L = 16
C = 80
CORE0_FRAC = 0.73
NIDX = 4
NBUF = 2


def _matmul_body(x_ref, w_ref, o_ref):
    o_ref[...] = jnp.dot(x_ref[...], w_ref[...], preferred_element_type=jnp.float32)


def _combine_body(p_ref, b_ref, o_ref):
    n = o_ref.shape[0]
    o_ref[...] = p_ref[0, :n] + p_ref[1, :n] + b_ref[...]


@functools.lru_cache(maxsize=None)
def _make_sc_scatter(n, d, ka, kb):
    assert ka % NIDX == 0 and kb % NIDX == 0 and min(ka, kb) >= NIDX
    rpt = n // NS
    nfull, rem = divmod(rpt, C)
    mesh = plsc.VectorSubcoreMesh(core_axis_name="c", subcore_axis_name="s")

    @functools.partial(
        pl.kernel,
        out_type=jax.ShapeDtypeStruct((NC, n, d), jnp.float32),
        mesh=mesh,
        scratch_types=[
            pltpu.VMEM((NIDX, 2, C), jnp.int32),
            pltpu.VMEM((NIDX, C), jnp.float32),
            pltpu.VMEM((NBUF, C, d), jnp.float32),
            pltpu.VMEM((NBUF, C, d), jnp.float32),
            pltpu.VMEM((NBUF, C), jnp.int32),
            pltpu.VMEM_SHARED((n, d), jnp.float32),
            [pltpu.SemaphoreType.DMA] * NIDX,
            [pltpu.SemaphoreType.DMA] * NIDX,
            [pltpu.SemaphoreType.DMA] * NBUF,
            [pltpu.SemaphoreType.DMA] * NBUF,
        ],
    )
    def sc_kernel(
        support, pk, wk, out, idx_v, w_v, gbuf, sbuf, dst_v, acc, sem_i, sem_w,
        sem_g, sem_s
    ):
        cid = lax.axis_index("c")
        sid = lax.axis_index("s")
        kc = jnp.where(cid == 0, ka, kb)
        row0 = jnp.where(cid == 0, sid * ka, NS * ka + sid * kb)

        def idx_start(b, j):
            pltpu.async_copy(pk.at[row0 + j], idx_v.at[b], sem_i[b])
            pltpu.async_copy(wk.at[row0 + j], w_v.at[b], sem_w[b])

        def idx_wait(b):
            pltpu.make_async_copy(pk.at[row0], idx_v.at[b], sem_i[b]).wait()
            pltpu.make_async_copy(wk.at[row0], w_v.at[b], sem_w[b]).wait()

        def gather_start(gs, b):
            pltpu.async_copy(support.at[idx_v.at[b, 0]], gbuf.at[gs], sem_g[gs])

        def gather_wait(gs):
            pltpu.make_async_copy(
                support.at[idx_v.at[0, 0]], gbuf.at[gs], sem_g[gs]
            ).wait()

        def scatter_start(gs):
            pltpu.async_copy(
                sbuf.at[gs], acc.at[dst_v.at[gs]], sem_s[gs], add=True
            )

        def scatter_wait(gs):
            pltpu.make_async_copy(
                sbuf.at[gs], acc.at[dst_v.at[gs]], sem_s[gs]
            ).wait()

        def scale(b, gs):
            @pl.loop(0, C // L)
            def _(blk):
                sl16 = pl.ds(blk * L, L)
                wv16 = w_v[b, sl16]
                dst_v[gs, sl16] = idx_v[b, 1, sl16]
                for r in range(L):
                    wvec = jnp.full((L,), wv16[r])
                    row = blk * L + r
                    for jj in range(d // L):
                        sl = pl.ds(jj * L, L)
                        sbuf[gs, row, sl] = gbuf[gs, row, sl] * wvec

        for b in range(NIDX):
            idx_start(b, b)

        zero = jnp.zeros((L,), jnp.float32)

        @pl.loop(0, C)
        def _(i):
            for j in range(d // L):
                gbuf[0, i, pl.ds(j * L, L)] = zero

        base = sid * rpt
        for t in range(nfull):
            pltpu.sync_copy(gbuf.at[0], acc.at[pl.ds(base + t * C, C)])
        if rem:
            pltpu.sync_copy(
                gbuf.at[0, pl.ds(0, rem)], acc.at[pl.ds(base + nfull * C, rem)]
            )
        plsc.subcore_barrier()

        for b in range(NBUF):
            idx_wait(b)
            gather_start(b, b)

        @pl.loop(0, kc, step=NIDX)
        def _(j0):
            for b in range(NIDX):
                j = j0 + b
                gs = b % NBUF
                gather_wait(gs)
                if b < NBUF:
                    @pl.when(j >= NBUF)
                    def _():
                        scatter_wait(gs)
                else:
                    scatter_wait(gs)
                scale(b, gs)
                scatter_start(gs)
                @pl.when(j + NBUF < kc)
                def _():
                    idx_wait((b + NBUF) % NIDX)
                    gather_start(gs, (b + NBUF) % NIDX)
                @pl.when(j + NIDX < kc)
                def _():
                    idx_start(b, j + NIDX)

        for gs in range(NBUF):
            scatter_wait(gs)
        plsc.subcore_barrier()

        for t in range(nfull):
            pltpu.sync_copy(
                acc.at[pl.ds(base + t * C, C)], out.at[cid, pl.ds(base + t * C, C)]
            )
        if rem:
            pltpu.sync_copy(
                acc.at[pl.ds(base + nfull * C, rem)],
                out.at[cid, pl.ds(base + nfull * C, rem)],
            )

    return sc_kernel


def kernel(input_feature, edge_index, edge_weight, weight, bias):
    n, _ = input_feature.shape
    d_out = weight.shape[1]
    e = edge_weight.shape[0]

    support = pl.pallas_call(
        _matmul_body,
        out_shape=jax.ShapeDtypeStruct((n, d_out), jnp.float32),
    )(input_feature, weight)

    nw = NC * NS
    k = -(-e // (nw * C))
    k = -(-k // NIDX) * NIDX
    ka = int(round(2 * k * CORE0_FRAC / NIDX)) * NIDX
    ka = min(max(ka, NIDX), 2 * k - NIDX)
    kb = 2 * k - ka
    epad = nw * k * C
    dst = edge_index[0].astype(jnp.int32)
    src = edge_index[1].astype(jnp.int32)
    ew = edge_weight.astype(jnp.float32)
    if epad != e:
        pad = epad - e
        dst = jnp.concatenate([dst, jnp.zeros((pad,), jnp.int32)])
        src = jnp.concatenate([src, jnp.zeros((pad,), jnp.int32)])
        ew = jnp.concatenate([ew, jnp.zeros((pad,), jnp.float32)])
    pk = jnp.concatenate(
        [src.reshape(nw * k, 1, C), dst.reshape(nw * k, 1, C)], axis=1
    )
    wk = ew.reshape(nw * k, C)

    rpt = -(-n // NS)
    rpt = (rpt + 7) // 8 * 8
    n_pad = rpt * NS

    partial = _make_sc_scatter(n_pad, d_out, ka, kb)(support, pk, wk)

    out = pl.pallas_call(
        _combine_body,
        out_shape=jax.ShapeDtypeStruct((n, d_out), jnp.float32),
    )(partial, bias.reshape(1, d_out))
    return out
